# baseline (speedup 1.0000x reference)
import functools

import jax
import jax.numpy as jnp
from jax import lax
from jax.experimental import pallas as pl
from jax.experimental.pallas import tpu as pltpu
from jax.experimental.pallas import tpu_sc as plsc

DEPTH = 64
WIDE = 2 * DEPTH
NC, NS = 2, 16
NW = NC * NS
LANES = 16
IDX_LANE = 128
K = 2
CHUNK = K * IDX_LANE
NBUF = 3


@functools.lru_cache(maxsize=None)
def _build_transpose(vocab):
    n_blocks = vocab // WIDE
    tail = vocab - n_blocks * WIDE
    iters = (n_blocks + NW - 1) // NW
    mesh = plsc.VectorSubcoreMesh(core_axis_name="c", subcore_axis_name="s")

    @functools.partial(
        pl.kernel,
        mesh=mesh,
        out_type=jax.ShapeDtypeStruct((vocab, WIDE), jnp.float32),
        scratch_types=[
            pltpu.VMEM((2, DEPTH, WIDE), jnp.float32),
            pltpu.VMEM((2, WIDE, WIDE), jnp.float32),
            [pltpu.SemaphoreType.DMA] * 2,
            [pltpu.SemaphoreType.DMA] * 2,
        ],
        compiler_params=pltpu.CompilerParams(
            use_tc_tiling_on_sc=True, needs_layout_passes=False
        ),
    )
    def body(tab_t, aux, out_hbm, in_v, out_v, sem_i, sem_o):
        wid = lax.axis_index("s") * NC + lax.axis_index("c")
        last = n_blocks - 1

        def blk(g):
            return jnp.minimum(wid + g * NW, last)

        def load(g, b):
            return pltpu.make_async_copy(
                tab_t.at[:, pl.ds(blk(g) * WIDE, WIDE)], in_v.at[b], sem_i[b]
            )

        def store(g, b):
            return pltpu.make_async_copy(
                out_v.at[b], out_hbm.at[pl.ds(blk(g) * WIDE, WIDE)], sem_o[b]
            )

        row_ids = [
            (jax.lax.iota(jnp.int32, LANES) + g * LANES) for g in range(4)
        ]

        def transpose(b):
            def rows(v0, carry):
                for u in range(8):
                    col = jnp.full((LANES,), v0 * 8 + u, jnp.int32)
                    for g in range(4):
                        vec = plsc.load_gather(in_v.at[b], [row_ids[g], col])
                        out_v[b, v0 * 8 + u, pl.ds(g * LANES, LANES)] = vec
                return carry

            lax.fori_loop(0, WIDE // 8, rows, 0)

        @pl.when(wid == 0)
        def _():
            if tail:
                pltpu.sync_copy(aux, in_v.at[0, pl.ds(0, tail)])
                pltpu.sync_copy(
                    in_v.at[0, pl.ds(0, tail)],
                    out_hbm.at[pl.ds(n_blocks * WIDE, tail)],
                )

        assert iters % 2 == 1 and iters >= 5
        load(0, 0).start()
        load(1, 1).start()
        load(0, 0).wait()
        transpose(0)
        store(0, 0).start()
        load(2, 0).start()
        load(1, 1).wait()
        transpose(1)
        store(1, 1).start()
        load(3, 1).start()

        def pair(p, carry):
            for b in (0, 1):
                i = 2 + p * 2 + b
                load(i, b).wait()
                store(i - 2, b).wait()
                transpose(b)
                store(i, b).start()

                @pl.when(i + 2 < iters)
                def _():
                    load(i + 2, b).start()

            return carry

        lax.fori_loop(0, (iters - 3) // 2, pair, 0)

        i_last = iters - 1
        load(i_last, 0).wait()
        store(i_last - 2, 0).wait()
        transpose(0)
        store(i_last, 0).start()
        store(i_last - 1, 1).wait()
        store(i_last, 0).wait()

    return body


@functools.lru_cache(maxsize=None)
def _build_gather(n_rows, vocab):
    assert n_rows % (NW * CHUNK) == 0
    n_chunks = n_rows // (NW * CHUNK)
    assert (n_chunks - 4) % NBUF == 0 and n_chunks >= 8
    idx_rows = n_chunks * K
    mesh = plsc.VectorSubcoreMesh(core_axis_name="c", subcore_axis_name="s")

    @functools.partial(
        pl.kernel,
        mesh=mesh,
        out_type=jax.ShapeDtypeStruct((n_rows, WIDE), jnp.float32),
        scratch_types=[
            pltpu.VMEM((idx_rows, IDX_LANE), jnp.int32),
            pltpu.VMEM((NBUF, CHUNK, WIDE), jnp.float32),
            [pltpu.SemaphoreType.DMA] * NBUF,
            [pltpu.SemaphoreType.DMA] * NBUF,
        ],
        compiler_params=pltpu.CompilerParams(use_tc_tiling_on_sc=True),
    )
    def body(table_hbm, idx_hbm, out_hbm, idx_v, rows_v, sem_g, sem_o):
        wid = lax.axis_index("s") * NC + lax.axis_index("c")
        chunk0 = wid * n_chunks

        pltpu.sync_copy(idx_hbm.at[pl.ds(chunk0 * K, idx_rows)], idx_v)

        def gather(i, b):
            return [
                pltpu.make_async_copy(
                    table_hbm.at[idx_v.at[i * K + j]],
                    rows_v.at[b, pl.ds(j * IDX_LANE, IDX_LANE)],
                    sem_g[b],
                )
                for j in range(K)
            ]

        def fire_g(i, b):
            for cp in gather(i, b):
                cp.start()

        def wait_g(i, b):
            for cp in gather(i, b):
                cp.wait()

        def store(i, b):
            return pltpu.make_async_copy(
                rows_v.at[b],
                out_hbm.at[pl.ds((chunk0 + i) * CHUNK, CHUNK)],
                sem_o[b],
            )

        fire_g(0, 0)
        fire_g(1, 1)
        wait_g(0, 0)
        store(0, 0).start()
        fire_g(2, 2)
        wait_g(1, 1)
        store(1, 1).start()
        store(0, 0).wait()
        fire_g(3, 0)

        def outer(g, carry):
            for k in range(NBUF):
                i = 2 + g * NBUF + k
                b = (2 + k) % NBUF
                wait_g(i, b)
                store(i, b).start()
                store(i - 1, (b + NBUF - 1) % NBUF).wait()
                fire_g(i + 2, (b + 2) % NBUF)
            return carry

        lax.fori_loop(0, (n_chunks - 4) // NBUF, outer, 0)

        i1, i2 = n_chunks - 2, n_chunks - 1
        b1, b2 = i1 % NBUF, i2 % NBUF
        wait_g(i1, b1)
        store(i1, b1).start()
        store(i1 - 1, (b1 + NBUF - 1) % NBUF).wait()
        wait_g(i2, b2)
        store(i2, b2).start()
        store(i1, b1).wait()
        store(i2, b2).wait()

    return body


def kernel(texts_tokenized, embedding_table):
    b, s = texts_tokenized.shape
    n_rows = b * s
    vocab = embedding_table.shape[0]
    tail = vocab % WIDE
    idx = texts_tokenized.reshape(n_rows // IDX_LANE, IDX_LANE).astype(jnp.int32)
    table_t = embedding_table.T
    aux = jnp.pad(embedding_table[vocab - tail:, :], ((0, 0), (0, WIDE - DEPTH)))
    scratch = _build_transpose(vocab)(table_t, aux)
    out = _build_gather(n_rows, vocab)(scratch, idx)
    return out[:, :DEPTH].reshape(b, s, DEPTH)


# speedup vs baseline: 1.9538x; 1.9538x over previous
import functools

import jax
import jax.numpy as jnp
from jax import lax
from jax.experimental import pallas as pl
from jax.experimental.pallas import tpu as pltpu
from jax.experimental.pallas import tpu_sc as plsc

DEPTH = 64
WIDE = 2 * DEPTH
NC, NS = 2, 16
NW = NC * NS
IDX_LANE = 128
K = 2
CHUNK = K * IDX_LANE
NBUF = 3


@functools.lru_cache(maxsize=None)
def _build(n_rows):
    assert n_rows % (NW * CHUNK) == 0
    n_chunks = n_rows // (NW * CHUNK)
    assert (n_chunks - 4) % NBUF == 0 and n_chunks >= 8
    idx_rows = n_chunks * K
    mesh = plsc.VectorSubcoreMesh(core_axis_name="c", subcore_axis_name="s")

    @functools.partial(
        pl.kernel,
        mesh=mesh,
        out_type=jax.ShapeDtypeStruct((n_rows, WIDE), jnp.float32),
        scratch_types=[
            pltpu.VMEM((idx_rows, IDX_LANE), jnp.int32),
            pltpu.VMEM((NBUF, CHUNK, WIDE), jnp.float32),
            [pltpu.SemaphoreType.DMA] * NBUF,
            [pltpu.SemaphoreType.DMA] * NBUF,
        ],
        compiler_params=pltpu.CompilerParams(use_tc_tiling_on_sc=True),
    )
    def body(table_hbm, idx_hbm, out_hbm, idx_v, rows_v, sem_g, sem_o):
        wid = lax.axis_index("s") * NC + lax.axis_index("c")
        chunk0 = wid * n_chunks

        pltpu.sync_copy(idx_hbm.at[pl.ds(chunk0 * K, idx_rows)], idx_v)

        def gather(i, b):
            return [
                pltpu.make_async_copy(
                    table_hbm.at[idx_v.at[i * K + j]],
                    rows_v.at[b, pl.ds(j * IDX_LANE, IDX_LANE)],
                    sem_g[b],
                )
                for j in range(K)
            ]

        def fire_g(i, b):
            for cp in gather(i, b):
                cp.start()

        def wait_g(i, b):
            for cp in gather(i, b):
                cp.wait()

        def store(i, b):
            return pltpu.make_async_copy(
                rows_v.at[b],
                out_hbm.at[pl.ds((chunk0 + i) * CHUNK, CHUNK)],
                sem_o[b],
            )

        fire_g(0, 0)
        fire_g(1, 1)
        wait_g(0, 0)
        store(0, 0).start()
        fire_g(2, 2)
        wait_g(1, 1)
        store(1, 1).start()
        store(0, 0).wait()
        fire_g(3, 0)

        def outer(g, carry):
            for k in range(NBUF):
                i = 2 + g * NBUF + k
                b = (2 + k) % NBUF
                wait_g(i, b)
                store(i, b).start()
                store(i - 1, (b + NBUF - 1) % NBUF).wait()
                fire_g(i + 2, (b + 2) % NBUF)
            return carry

        lax.fori_loop(0, (n_chunks - 4) // NBUF, outer, 0)

        i1, i2 = n_chunks - 2, n_chunks - 1
        b1, b2 = i1 % NBUF, i2 % NBUF
        wait_g(i1, b1)
        store(i1, b1).start()
        store(i1 - 1, (b1 + NBUF - 1) % NBUF).wait()
        wait_g(i2, b2)
        store(i2, b2).start()
        store(i1, b1).wait()
        store(i2, b2).wait()

    return body


def kernel(texts_tokenized, embedding_table):
    b, s = texts_tokenized.shape
    n_rows = b * s
    idx = texts_tokenized.reshape(n_rows // IDX_LANE, IDX_LANE).astype(jnp.int32)
    table_p = jnp.pad(embedding_table, ((0, 0), (0, WIDE - DEPTH)))
    out = _build(n_rows)(table_p, idx)
    return out[:, :DEPTH].reshape(b, s, DEPTH)
